# baseline (speedup 1.0000x reference)
import functools

import jax
import jax.numpy as jnp
from jax.experimental import pallas as pl

TI = 128


def _bf16_round(v):
    u = jax.lax.bitcast_convert_type(v, jnp.uint32)
    r = (u + jnp.uint32(0x7FFF) + ((u >> 16) & jnp.uint32(1))) & jnp.uint32(0xFFFF0000)
    return jax.lax.bitcast_convert_type(r, jnp.float32)


def _layernorm(h, g, b):
    m = h.mean(-1, keepdims=True)
    v = h.var(-1, keepdims=True)
    return (h - m) / jnp.sqrt(v + 1e-5) * g + b


def _mlp(x, Ws, bs_, gs, betas):
    h = x
    for i in range(3):
        h = _layernorm(h, gs[i], betas[i])
        h = jnp.dot(h, Ws[i], preferred_element_type=jnp.float32) + bs_[i]
        if i < 2:
            h = 0.5 * h * (1.0 + jax.lax.erf(h * 0.7071067811865476))
    return h


def _body(hm_ref, amask_ref, wh_ref, wk_ref, w10_ref, b10_ref, w11_ref,
          b11_ref, w12_ref, b12_ref, w20_ref, b20_ref, w21_ref, b21_ref,
          w22_ref, b22_ref, g0_ref, be0_ref, g1_ref, be1_ref, g2_ref,
          be2_ref, bh_bk_ref, out_ref):
    wv = _bf16_round(wh_ref[...]).reshape(4, 16)
    acc = jnp.zeros((TI, 588), jnp.float32)
    for l in range(4):
        for h in range(16):
            p = _bf16_round(hm_ref[l, 0, h])
            acc = acc + p * wv[l, h]
    x = acc + bh_bk_ref[0, 0]

    xb = _bf16_round(x)
    wkb = _bf16_round(wk_ref[...]).reshape(1, 588)
    logits = jnp.sum(xb * wkb, axis=1, keepdims=True)
    logits = logits + bh_bk_ref[0, 1]
    mask = logits > 0.0

    gs = (g0_ref[...], g1_ref[...], g2_ref[...])
    betas = (be0_ref[...], be1_ref[...], be2_ref[...])
    p1 = _mlp(x, (w10_ref[...], w11_ref[...], w12_ref[...]),
              (b10_ref[...], b11_ref[...], b12_ref[...]), gs, betas)
    p2 = _mlp(x, (w20_ref[...], w21_ref[...], w22_ref[...]),
              (b20_ref[...], b21_ref[...], b22_ref[...]), gs, betas)
    out = jnp.where(mask, p1, p2)
    out = jax.nn.sigmoid(out) * amask_ref[...]

    x1 = out[:, 0:1]
    y1 = out[:, 1:2]
    x2 = x1 + out[:, 2:3]
    y2 = y1 + out[:, 3:4]
    zeros = jnp.zeros((TI, 3), jnp.float32)
    out_ref[...] = jnp.concatenate([x1, y1, x2, y2, logits, zeros], axis=1)


def kernel(heatmap, attention_valid_mask, Wh, bh, Wk, bk,
           W1_0, b1_0, W1_1, b1_1, W1_2, b1_2,
           W2_0, b2_0, W2_1, b2_1, W2_2, b2_2,
           g_0, beta_0, g_1, beta_1, g_2, beta_2):
    num_layer, bs, num_heads, input_len, encoder_len = heatmap.shape
    nt = input_len // TI
    rows = bs * input_len

    amask = attention_valid_mask.reshape(rows, 1)
    whr = Wh.reshape(1, 64)
    bh_bk = jnp.stack([bh[0], bk[0]]).reshape(1, 2)

    def rep(_b, _t):
        return (0, 0)

    grid = (bs, nt)
    out_all = pl.pallas_call(
        _body,
        grid=grid,
        in_specs=[
            pl.BlockSpec((num_layer, 1, num_heads, TI, encoder_len),
                         lambda b, t: (0, b, 0, t, 0)),
            pl.BlockSpec((TI, 1), lambda b, t: (b * nt + t, 0)),
            pl.BlockSpec((1, 64), rep),
            pl.BlockSpec((encoder_len, 1), rep),
            pl.BlockSpec(W1_0.shape, rep), pl.BlockSpec((1, 256), rep),
            pl.BlockSpec(W1_1.shape, rep), pl.BlockSpec((1, 256), rep),
            pl.BlockSpec(W1_2.shape, rep), pl.BlockSpec((1, 4), rep),
            pl.BlockSpec(W2_0.shape, rep), pl.BlockSpec((1, 256), rep),
            pl.BlockSpec(W2_1.shape, rep), pl.BlockSpec((1, 256), rep),
            pl.BlockSpec(W2_2.shape, rep), pl.BlockSpec((1, 4), rep),
            pl.BlockSpec((1, 588), rep), pl.BlockSpec((1, 588), rep),
            pl.BlockSpec((1, 256), rep), pl.BlockSpec((1, 256), rep),
            pl.BlockSpec((1, 256), rep), pl.BlockSpec((1, 256), rep),
            pl.BlockSpec((1, 2), rep),
        ],
        out_specs=pl.BlockSpec((TI, 8), lambda b, t: (b * nt + t, 0)),
        out_shape=jax.ShapeDtypeStruct((rows, 8), jnp.float32),
    )(heatmap, amask, whr, Wk,
      W1_0, b1_0.reshape(1, 256), W1_1, b1_1.reshape(1, 256),
      W1_2, b1_2.reshape(1, 4),
      W2_0, b2_0.reshape(1, 256), W2_1, b2_1.reshape(1, 256),
      W2_2, b2_2.reshape(1, 4),
      g_0.reshape(1, 588), beta_0.reshape(1, 588),
      g_1.reshape(1, 256), beta_1.reshape(1, 256),
      g_2.reshape(1, 256), beta_2.reshape(1, 256),
      bh_bk)

    o = out_all.reshape(bs, input_len, 8)
    return (o[:, :, 0], o[:, :, 1], o[:, :, 2], o[:, :, 3],
            out_all[:, 4])
